# baseline (speedup 1.0000x reference)
import jax
import jax.numpy as jnp
from jax import lax
from jax.experimental import pallas as pl
from jax.experimental.pallas import tpu as pltpu
from jax.experimental.pallas import tpu_sc as plsc

N = 10000
E = 160000
D = 256
C = 256
H = 4

EPAD = 172032
NROW = 10112
RW = 632
B = 128
NB1 = EPAD // 32 // B
B2 = 96
NB2 = EPAD // 16 // B2
Q = 64

_BLK_N = 1000
_GRID = N // _BLK_N


def _dense_body(x_ref, wcat_ref, ams_ref, amd_ref,
                y_ref, xh4_ref, as_ref, ad_ref, shift_ref,
                mxs_ref, mxd_ref):
    i = pl.program_id(0)
    y = jnp.dot(x_ref[...], wcat_ref[...], preferred_element_type=jnp.float32)
    y_ref[...] = y
    xh = y[:, : H * C]
    for k in range(4):
        lo = (k // 2) * 128 + (k % 2) * Q
        xh4_ref[k] = jnp.concatenate(
            [xh[:, h * C + lo: h * C + lo + Q] for h in range(H)], axis=-1)
    a_s = jnp.dot(xh, ams_ref[...], preferred_element_type=jnp.float32)
    a_d = jnp.dot(xh, amd_ref[...], preferred_element_type=jnp.float32)
    as_ref[...] = a_s
    ad_ref[...] = a_d

    @pl.when(i == 0)
    def _():
        mxs_ref[...] = jnp.full((8, 16), -1e30, jnp.float32)
        mxd_ref[...] = jnp.full((8, 16), -1e30, jnp.float32)

    mxs_ref[...] = jnp.maximum(mxs_ref[...],
                               jnp.max(a_s, axis=0, keepdims=True))
    mxd_ref[...] = jnp.maximum(mxd_ref[...],
                               jnp.max(a_d, axis=0, keepdims=True))

    @pl.when(i == _GRID - 1)
    def _():
        shift_ref[...] = jnp.maximum(mxs_ref[...] + mxd_ref[...], 0.0)


def _dense_phase(x, wcat, attm_s, attm_d):
    return pl.pallas_call(
        _dense_body,
        grid=(_GRID,),
        in_specs=[
            pl.BlockSpec((_BLK_N, D), lambda i: (i, 0)),
            pl.BlockSpec((D, H * C + C), lambda i: (0, 0)),
            pl.BlockSpec((H * C, 16), lambda i: (0, 0)),
            pl.BlockSpec((H * C, 16), lambda i: (0, 0)),
        ],
        out_specs=[
            pl.BlockSpec((_BLK_N, H * C + C), lambda i: (i, 0)),
            pl.BlockSpec((4, _BLK_N, H * Q), lambda i: (0, i, 0)),
            pl.BlockSpec((_BLK_N, 16), lambda i: (i, 0)),
            pl.BlockSpec((_BLK_N, 16), lambda i: (i, 0)),
            pl.BlockSpec((8, 16), lambda i: (0, 0)),
        ],
        out_shape=[
            jax.ShapeDtypeStruct((N, H * C + C), jnp.float32),
            jax.ShapeDtypeStruct((4, N, H * Q), jnp.float32),
            jax.ShapeDtypeStruct((NROW, 16), jnp.float32),
            jax.ShapeDtypeStruct((NROW, 16), jnp.float32),
            jax.ShapeDtypeStruct((8, 16), jnp.float32),
        ],
        scratch_shapes=[
            pltpu.VMEM((8, 16), jnp.float32),
            pltpu.VMEM((8, 16), jnp.float32),
        ],
    )(x, wcat, attm_s, attm_d)


_MESH = plsc.VectorSubcoreMesh(core_axis_name="c", subcore_axis_name="s")


def _pass1_body(src_hbm, dst_hbm, as_hbm, ad_hbm, shift_hbm,
                w_hbm, dpart_hbm,
                sidx_v, didx_v, gs_v, gd_v, w_v, shift_v, row_v,
                denom_sh, sem):
    cid = lax.axis_index("c")
    sid = lax.axis_index("s")
    wid = sid * 2 + cid
    r0 = sid * RW

    def _zrow(e, _):
        row_v[e] = jnp.zeros((16,), jnp.float32)
        return _
    lax.fori_loop(0, RW, _zrow, None)
    pltpu.sync_copy(row_v, denom_sh.at[pl.ds(r0, RW)])
    plsc.subcore_barrier()

    pltpu.sync_copy(shift_hbm, shift_v)
    shift = shift_v[0]

    def _blk(b, _):
        base = wid * (NB1 * B) + b * B
        pltpu.sync_copy(src_hbm.at[pl.ds(base, B)], sidx_v)
        pltpu.sync_copy(dst_hbm.at[pl.ds(base, B)], didx_v)
        pltpu.async_copy(as_hbm.at[sidx_v], gs_v, sem).wait()
        pltpu.async_copy(ad_hbm.at[didx_v], gd_v, sem).wait()

        def _edge(e, _):
            t = gs_v[e] + gd_v[e]
            t = jnp.where(t >= 0.0, t, 0.2 * t)
            w_v[e] = jnp.exp(t - shift)
            return _
        lax.fori_loop(0, B, _edge, None)

        pltpu.sync_copy(w_v, w_hbm.at[pl.ds(base, B)])
        pltpu.sync_copy(w_v, denom_sh.at[didx_v], add=True)
        return _
    lax.fori_loop(0, NB1, _blk, None)

    plsc.subcore_barrier()
    pltpu.sync_copy(denom_sh.at[pl.ds(r0, RW)], row_v)
    pltpu.sync_copy(row_v, dpart_hbm.at[cid, pl.ds(r0, RW)])


def _pass1(src_e, dst_e, as_p, ad_p, shift):
    f = pl.kernel(
        _pass1_body,
        out_type=[
            jax.ShapeDtypeStruct((EPAD, 16), jnp.float32),
            jax.ShapeDtypeStruct((2, NROW, 16), jnp.float32),
        ],
        mesh=_MESH,
        scratch_types=[
            pltpu.VMEM((B,), jnp.int32),
            pltpu.VMEM((B,), jnp.int32),
            pltpu.VMEM((B, 16), jnp.float32),
            pltpu.VMEM((B, 16), jnp.float32),
            pltpu.VMEM((B, 16), jnp.float32),
            pltpu.VMEM((8, 16), jnp.float32),
            pltpu.VMEM((RW, 16), jnp.float32),
            pltpu.VMEM_SHARED((NROW, 16), jnp.float32),
            pltpu.SemaphoreType.DMA,
        ],
        compiler_params=pltpu.CompilerParams(use_tc_tiling_on_sc=False),
    )
    return f(src_e, dst_e, as_p, ad_p, shift)


def _pass2_body(src_hbm, dst_hbm, w_hbm, dflat_hbm, xh_hbm,
                conv_hbm,
                sraw_v, draw_v, didx2_v, dscat_v,
                z_v, w_v, d0_v, d1_v, msg_v,
                conv_sh, gsem0, gsem1, isem0, isem1, ssem):
    cid = lax.axis_index("c")
    sid = lax.axis_index("s")
    r0 = sid * RW
    ch0 = sid * (NB2 * B2)
    gsems = (gsem0, gsem1)
    isems = (isem0, isem1)

    def _fire_idx(b, k):
        base = ch0 + jnp.minimum(b, NB2 - 1) * B2
        pltpu.async_copy(src_hbm.at[pl.ds(base, B2)], sraw_v.at[k], isems[k])
        pltpu.async_copy(dst_hbm.at[pl.ds(base, B2)], draw_v.at[k], isems[k])

    def _wait_idx(k):
        pltpu.make_async_copy(
            src_hbm.at[pl.ds(0, B2)], sraw_v.at[k], isems[k]).wait()
        pltpu.make_async_copy(
            dst_hbm.at[pl.ds(0, B2)], draw_v.at[k], isems[k]).wait()

    def _fire_gathers(b, k, coff):
        def _bld(j, _):
            sraw_v[k, pl.ds(j * 16, 16)] = sraw_v[k, pl.ds(j * 16, 16)] + coff
            didx2_v[k, pl.ds(j * 16, 16)] = draw_v[k, pl.ds(j * 16, 16)] + NROW
            return _
        lax.fori_loop(0, B2 // 16, _bld, None)
        base = ch0 + jnp.minimum(b, NB2 - 1) * B2
        pltpu.async_copy(xh_hbm.at[sraw_v.at[k]], z_v.at[k], gsems[k])
        pltpu.async_copy(w_hbm.at[pl.ds(base, B2)], w_v.at[k], gsems[k])
        pltpu.async_copy(dflat_hbm.at[draw_v.at[k]], d0_v.at[k], gsems[k])
        pltpu.async_copy(dflat_hbm.at[didx2_v.at[k]], d1_v.at[k], gsems[k])

    def _wait_gathers(k):
        pltpu.make_async_copy(
            xh_hbm.at[sraw_v.at[k]], z_v.at[k], gsems[k]).wait()
        pltpu.make_async_copy(
            w_hbm.at[pl.ds(0, B2)], w_v.at[k], gsems[k]).wait()
        pltpu.make_async_copy(
            dflat_hbm.at[draw_v.at[k]], d0_v.at[k], gsems[k]).wait()
        pltpu.make_async_copy(
            dflat_hbm.at[didx2_v.at[k]], d1_v.at[k], gsems[k]).wait()

    def _mkdscat(k):
        def _cp(j, _):
            dscat_v[k, pl.ds(j * 16, 16)] = draw_v[k, pl.ds(j * 16, 16)]
            return _
        lax.fori_loop(0, B2 // 16, _cp, None)

    def _fma_scatter(k):
        def _fma(e, _):
            d = d0_v[k, e] + d1_v[k, e] + 1e-16
            al = w_v[k, e] / d
            a0 = al[0]
            a1 = al[1]
            a2 = al[2]
            a3 = al[3]
            for j in range(Q // 16):
                acc = a0 * z_v[k, e, pl.ds(j * 16, 16)]
                acc = acc + a1 * z_v[k, e, pl.ds(Q + j * 16, 16)]
                acc = acc + a2 * z_v[k, e, pl.ds(2 * Q + j * 16, 16)]
                acc = acc + a3 * z_v[k, e, pl.ds(3 * Q + j * 16, 16)]
                msg_v[k, e, pl.ds(j * 16, 16)] = acc
            return _
        lax.fori_loop(0, B2, _fma, None)
        pltpu.make_async_copy(
            msg_v.at[1 - k], conv_sh.at[dscat_v.at[1 - k]], ssem).wait()
        pltpu.async_copy(
            msg_v.at[k], conv_sh.at[dscat_v.at[k]], ssem, add=True)

    def _slot(b, k, coff):
        _wait_idx(1 - k)
        _fire_gathers(b + 1, 1 - k, coff)
        _wait_gathers(k)
        _mkdscat(k)
        _fire_idx(b + 2, k)
        _fma_scatter(k)

    for q in range(2):
        def _zrow(e, _):
            for j in range(Q // 16):
                msg_v[0, e, pl.ds(j * 16, 16)] = jnp.zeros((16,), jnp.float32)
                msg_v[1, e, pl.ds(j * 16, 16)] = jnp.zeros((16,), jnp.float32)
            return _
        lax.fori_loop(0, B2, _zrow, None)
        for k in range(6):
            pltpu.sync_copy(msg_v.at[0], conv_sh.at[pl.ds(r0 + k * B2, B2)])
        pltpu.sync_copy(msg_v.at[0, pl.ds(0, RW - 6 * B2)],
                        conv_sh.at[pl.ds(r0 + 6 * B2, RW - 6 * B2)])
        plsc.subcore_barrier()

        coff = (cid * 2 + q) * N

        def _zds(j, _):
            dscat_v[1, pl.ds(j * 16, 16)] = jnp.zeros((16,), jnp.int32)
            return _
        lax.fori_loop(0, B2 // 16, _zds, None)
        pltpu.async_copy(msg_v.at[1], conv_sh.at[dscat_v.at[1]], ssem,
                         add=True)

        _fire_idx(0, 0)
        _wait_idx(0)
        _fire_gathers(0, 0, coff)
        _fire_idx(1, 1)

        def _pair(p, _):
            _slot(2 * p, 0, coff)
            _slot(2 * p + 1, 1, coff)
            return _
        lax.fori_loop(0, NB2 // 2, _pair, None)

        _wait_gathers(0)
        _wait_idx(1)
        pltpu.make_async_copy(
            msg_v.at[1], conv_sh.at[dscat_v.at[1]], ssem).wait()

        plsc.subcore_barrier()
        for k in range(6):
            pltpu.sync_copy(conv_sh.at[pl.ds(r0 + k * B2, B2)], msg_v.at[0])
            pltpu.sync_copy(msg_v.at[0],
                            conv_hbm.at[cid, q, pl.ds(r0 + k * B2, B2)])
        pltpu.sync_copy(conv_sh.at[pl.ds(r0 + 6 * B2, RW - 6 * B2)],
                        msg_v.at[0, pl.ds(0, RW - 6 * B2)])
        pltpu.sync_copy(msg_v.at[0, pl.ds(0, RW - 6 * B2)],
                        conv_hbm.at[cid, q, pl.ds(r0 + 6 * B2, RW - 6 * B2)])
        plsc.subcore_barrier()


def _pass2(src_e, dst_e, w_t, dflat, xh_flat):
    f = pl.kernel(
        _pass2_body,
        out_type=jax.ShapeDtypeStruct((2, 2, NROW, Q), jnp.float32),
        mesh=_MESH,
        scratch_types=[
            pltpu.VMEM((2, B2), jnp.int32),
            pltpu.VMEM((2, B2), jnp.int32),
            pltpu.VMEM((2, B2), jnp.int32),
            pltpu.VMEM((2, B2), jnp.int32),
            pltpu.VMEM((2, B2, H * Q), jnp.float32),
            pltpu.VMEM((2, B2, 16), jnp.float32),
            pltpu.VMEM((2, B2, 16), jnp.float32),
            pltpu.VMEM((2, B2, 16), jnp.float32),
            pltpu.VMEM((2, B2, Q), jnp.float32),
            pltpu.VMEM_SHARED((NROW, Q), jnp.float32),
            pltpu.SemaphoreType.DMA,
            pltpu.SemaphoreType.DMA,
            pltpu.SemaphoreType.DMA,
            pltpu.SemaphoreType.DMA,
            pltpu.SemaphoreType.DMA,
        ],
        compiler_params=pltpu.CompilerParams(use_tc_tiling_on_sc=False),
    )
    return f(src_e, dst_e, w_t, dflat, xh_flat)


def _final_body(c0_ref, c1_ref, c2_ref, c3_ref, y_ref, bias_ref, out_ref):
    conv = jnp.concatenate(
        [c0_ref[...], c1_ref[...], c2_ref[...], c3_ref[...]],
        axis=-1) * (1.0 / H)
    out_ref[...] = jnp.maximum(conv + bias_ref[...] + y_ref[...], 0.0)


def _final(conv_p, y_out, bias):
    return pl.pallas_call(
        _final_body,
        grid=(_GRID,),
        in_specs=[
            pl.BlockSpec((_BLK_N, Q), lambda i: (i, 0)),
            pl.BlockSpec((_BLK_N, Q), lambda i: (i, 0)),
            pl.BlockSpec((_BLK_N, Q), lambda i: (i, 0)),
            pl.BlockSpec((_BLK_N, Q), lambda i: (i, 0)),
            pl.BlockSpec((_BLK_N, C), lambda i: (i, 4)),
            pl.BlockSpec((1, C), lambda i: (0, 0)),
        ],
        out_specs=pl.BlockSpec((_BLK_N, C), lambda i: (i, 0)),
        out_shape=jax.ShapeDtypeStruct((N, C), jnp.float32),
    )(conv_p[0, 0], conv_p[0, 1], conv_p[1, 0], conv_p[1, 1],
      y_out, bias.reshape(1, C))


def kernel(x, edge_index, W, att_src, att_dst, bias, W_lin):
    wcat = jnp.concatenate([W, W_lin.T], axis=1)
    hsel = (jnp.arange(16)[None, :] == (jnp.arange(H * C) // C)[:, None])
    attm_s = jnp.where(hsel, att_src.reshape(H * C)[:, None], 0.0)
    attm_d = jnp.where(hsel, att_dst.reshape(H * C)[:, None], 0.0)

    y_out, xh4, as_t, ad_t, shift = _dense_phase(x, wcat, attm_s, attm_d)

    as_p, ad_p = as_t, ad_t
    xh_flat = xh4.reshape(4 * N, H * Q)

    pad = EPAD - E - N
    loop = jnp.arange(N, dtype=jnp.int32)
    src_e = jnp.concatenate(
        [edge_index[0], loop, jnp.zeros((pad,), jnp.int32)])
    dst_e = jnp.concatenate(
        [edge_index[1], loop, jnp.full((pad,), N, jnp.int32)])

    w_t, dpart = _pass1(src_e, dst_e, as_p, ad_p, shift)
    dflat = dpart.reshape(2 * NROW, 16)
    conv_p = _pass2(src_e, dst_e, w_t, dflat, xh_flat)
    return _final(conv_p, y_out, bias)


# speedup vs baseline: 1.0906x; 1.0906x over previous
import jax
import jax.numpy as jnp
from jax import lax
from jax.experimental import pallas as pl
from jax.experimental.pallas import tpu as pltpu
from jax.experimental.pallas import tpu_sc as plsc

N = 10000
E = 160000
D = 256
C = 256
H = 4

EPAD = 172032
NROW = 10112
RW = 632
B = 128
NB1 = EPAD // 32 // B
B2 = 96
NB2 = EPAD // 16 // B2
Q = 64

_BLK_N = 1000
_GRID = N // _BLK_N


def _dense_body(x_ref, wcat_ref, ams_ref, amd_ref,
                y_ref, xh4_ref, as_ref, ad_ref, shift_ref,
                mxs_ref, mxd_ref):
    i = pl.program_id(0)
    y = jnp.dot(x_ref[...], wcat_ref[...], preferred_element_type=jnp.float32)
    y_ref[...] = y
    xh = y[:, : H * C]
    for k in range(4):
        lo = (k // 2) * 128 + (k % 2) * Q
        xh4_ref[k] = jnp.concatenate(
            [xh[:, h * C + lo: h * C + lo + Q] for h in range(H)], axis=-1)
    a_s = jnp.dot(xh, ams_ref[...], preferred_element_type=jnp.float32)
    a_d = jnp.dot(xh, amd_ref[...], preferred_element_type=jnp.float32)
    as_ref[...] = a_s
    ad_ref[...] = a_d

    @pl.when(i == 0)
    def _():
        mxs_ref[...] = jnp.full((8, 16), -1e30, jnp.float32)
        mxd_ref[...] = jnp.full((8, 16), -1e30, jnp.float32)

    mxs_ref[...] = jnp.maximum(mxs_ref[...],
                               jnp.max(a_s, axis=0, keepdims=True))
    mxd_ref[...] = jnp.maximum(mxd_ref[...],
                               jnp.max(a_d, axis=0, keepdims=True))

    @pl.when(i == _GRID - 1)
    def _():
        shift_ref[...] = jnp.maximum(mxs_ref[...] + mxd_ref[...], 0.0)


def _dense_phase(x, wcat, attm_s, attm_d):
    return pl.pallas_call(
        _dense_body,
        grid=(_GRID,),
        in_specs=[
            pl.BlockSpec((_BLK_N, D), lambda i: (i, 0)),
            pl.BlockSpec((D, H * C + C), lambda i: (0, 0)),
            pl.BlockSpec((H * C, 16), lambda i: (0, 0)),
            pl.BlockSpec((H * C, 16), lambda i: (0, 0)),
        ],
        out_specs=[
            pl.BlockSpec((_BLK_N, H * C + C), lambda i: (i, 0)),
            pl.BlockSpec((4, _BLK_N, H * Q), lambda i: (0, i, 0)),
            pl.BlockSpec((_BLK_N, 16), lambda i: (i, 0)),
            pl.BlockSpec((_BLK_N, 16), lambda i: (i, 0)),
            pl.BlockSpec((8, 16), lambda i: (0, 0)),
        ],
        out_shape=[
            jax.ShapeDtypeStruct((N, H * C + C), jnp.float32),
            jax.ShapeDtypeStruct((4, N, H * Q), jnp.float32),
            jax.ShapeDtypeStruct((NROW, 16), jnp.float32),
            jax.ShapeDtypeStruct((NROW, 16), jnp.float32),
            jax.ShapeDtypeStruct((8, 16), jnp.float32),
        ],
        scratch_shapes=[
            pltpu.VMEM((8, 16), jnp.float32),
            pltpu.VMEM((8, 16), jnp.float32),
        ],
    )(x, wcat, attm_s, attm_d)


_MESH = plsc.VectorSubcoreMesh(core_axis_name="c", subcore_axis_name="s")


def _pass1_body(src_hbm, dst_hbm, as_hbm, ad_hbm, shift_hbm,
                w_hbm, dpart_hbm,
                sidx_v, didx_v, gs_v, gd_v, w_v, shift_v, row_v,
                denom_sh, sem):
    cid = lax.axis_index("c")
    sid = lax.axis_index("s")
    wid = sid * 2 + cid
    r0 = sid * RW

    def _zrow(e, _):
        row_v[e] = jnp.zeros((16,), jnp.float32)
        return _
    lax.fori_loop(0, RW, _zrow, None)
    pltpu.sync_copy(row_v, denom_sh.at[pl.ds(r0, RW)])
    plsc.subcore_barrier()

    pltpu.sync_copy(shift_hbm, shift_v)
    shift = shift_v[0]

    def _blk(b, _):
        base = wid * (NB1 * B) + b * B
        pltpu.sync_copy(src_hbm.at[pl.ds(base, B)], sidx_v)
        pltpu.sync_copy(dst_hbm.at[pl.ds(base, B)], didx_v)
        pltpu.async_copy(as_hbm.at[sidx_v], gs_v, sem).wait()
        pltpu.async_copy(ad_hbm.at[didx_v], gd_v, sem).wait()

        def _edge(e, _):
            t = gs_v[e] + gd_v[e]
            t = jnp.where(t >= 0.0, t, 0.2 * t)
            w_v[e] = jnp.exp(t - shift)
            return _
        lax.fori_loop(0, B, _edge, None)

        pltpu.sync_copy(w_v, w_hbm.at[pl.ds(base, B)])
        pltpu.sync_copy(w_v, denom_sh.at[didx_v], add=True)
        return _
    lax.fori_loop(0, NB1, _blk, None)

    plsc.subcore_barrier()
    pltpu.sync_copy(denom_sh.at[pl.ds(r0, RW)], row_v)
    pltpu.sync_copy(row_v, dpart_hbm.at[cid, pl.ds(r0, RW)])


def _pass1(src_e, dst_e, as_p, ad_p, shift):
    f = pl.kernel(
        _pass1_body,
        out_type=[
            jax.ShapeDtypeStruct((EPAD, 16), jnp.float32),
            jax.ShapeDtypeStruct((2, NROW, 16), jnp.float32),
        ],
        mesh=_MESH,
        scratch_types=[
            pltpu.VMEM((B,), jnp.int32),
            pltpu.VMEM((B,), jnp.int32),
            pltpu.VMEM((B, 16), jnp.float32),
            pltpu.VMEM((B, 16), jnp.float32),
            pltpu.VMEM((B, 16), jnp.float32),
            pltpu.VMEM((8, 16), jnp.float32),
            pltpu.VMEM((RW, 16), jnp.float32),
            pltpu.VMEM_SHARED((NROW, 16), jnp.float32),
            pltpu.SemaphoreType.DMA,
        ],
        compiler_params=pltpu.CompilerParams(use_tc_tiling_on_sc=False),
    )
    return f(src_e, dst_e, as_p, ad_p, shift)


def _dsum_body(dp_ref, rcp_ref):
    rcp_ref[...] = 1.0 / (dp_ref[0] + dp_ref[1] + 1e-16)


def _dsum(dpart):
    return pl.pallas_call(
        _dsum_body,
        out_shape=jax.ShapeDtypeStruct((NROW, 16), jnp.float32),
    )(dpart)


def _pass2_body(src_hbm, dst_hbm, w_hbm, rcp_hbm, xh_hbm,
                conv_hbm,
                sraw_v, draw_v, dscat_v,
                z_v, w_v, d0_v, msg_v,
                conv_sh, gsem0, gsem1, isem0, isem1, ssem):
    cid = lax.axis_index("c")
    sid = lax.axis_index("s")
    r0 = sid * RW
    ch0 = sid * (NB2 * B2)
    gsems = (gsem0, gsem1)
    isems = (isem0, isem1)

    def _fire_idx(b, k):
        base = ch0 + jnp.minimum(b, NB2 - 1) * B2
        pltpu.async_copy(src_hbm.at[pl.ds(base, B2)], sraw_v.at[k], isems[k])
        pltpu.async_copy(dst_hbm.at[pl.ds(base, B2)], draw_v.at[k], isems[k])

    def _wait_idx(k):
        pltpu.make_async_copy(
            src_hbm.at[pl.ds(0, B2)], sraw_v.at[k], isems[k]).wait()
        pltpu.make_async_copy(
            dst_hbm.at[pl.ds(0, B2)], draw_v.at[k], isems[k]).wait()

    def _fire_gathers(b, k, coff):
        def _bld(j, _):
            sraw_v[k, pl.ds(j * 16, 16)] = sraw_v[k, pl.ds(j * 16, 16)] + coff
            return _
        lax.fori_loop(0, B2 // 16, _bld, None)
        base = ch0 + jnp.minimum(b, NB2 - 1) * B2
        pltpu.async_copy(xh_hbm.at[sraw_v.at[k]], z_v.at[k], gsems[k])
        pltpu.async_copy(w_hbm.at[pl.ds(base, B2)], w_v.at[k], gsems[k])
        pltpu.async_copy(rcp_hbm.at[draw_v.at[k]], d0_v.at[k], gsems[k])

    def _wait_gathers(k):
        pltpu.make_async_copy(
            xh_hbm.at[sraw_v.at[k]], z_v.at[k], gsems[k]).wait()
        pltpu.make_async_copy(
            w_hbm.at[pl.ds(0, B2)], w_v.at[k], gsems[k]).wait()
        pltpu.make_async_copy(
            rcp_hbm.at[draw_v.at[k]], d0_v.at[k], gsems[k]).wait()

    def _mkdscat(k):
        def _cp(j, _):
            dscat_v[k, pl.ds(j * 16, 16)] = draw_v[k, pl.ds(j * 16, 16)]
            return _
        lax.fori_loop(0, B2 // 16, _cp, None)

    def _fma_scatter(k):
        def _fma(e, _):
            al = w_v[k, e] * d0_v[k, e]
            a0 = al[0]
            a1 = al[1]
            a2 = al[2]
            a3 = al[3]
            for j in range(Q // 16):
                acc = a0 * z_v[k, e, pl.ds(j * 16, 16)]
                acc = acc + a1 * z_v[k, e, pl.ds(Q + j * 16, 16)]
                acc = acc + a2 * z_v[k, e, pl.ds(2 * Q + j * 16, 16)]
                acc = acc + a3 * z_v[k, e, pl.ds(3 * Q + j * 16, 16)]
                msg_v[k, e, pl.ds(j * 16, 16)] = acc
            return _
        lax.fori_loop(0, B2, _fma, None)
        pltpu.make_async_copy(
            msg_v.at[1 - k], conv_sh.at[dscat_v.at[1 - k]], ssem).wait()
        pltpu.async_copy(
            msg_v.at[k], conv_sh.at[dscat_v.at[k]], ssem, add=True)

    def _slot(b, k, coff):
        _wait_idx(1 - k)
        _fire_gathers(b + 1, 1 - k, coff)
        _wait_gathers(k)
        _mkdscat(k)
        _fire_idx(b + 2, k)
        _fma_scatter(k)

    for q in range(2):
        def _zrow(e, _):
            for j in range(Q // 16):
                msg_v[0, e, pl.ds(j * 16, 16)] = jnp.zeros((16,), jnp.float32)
                msg_v[1, e, pl.ds(j * 16, 16)] = jnp.zeros((16,), jnp.float32)
            return _
        lax.fori_loop(0, B2, _zrow, None)
        for k in range(6):
            pltpu.sync_copy(msg_v.at[0], conv_sh.at[pl.ds(r0 + k * B2, B2)])
        pltpu.sync_copy(msg_v.at[0, pl.ds(0, RW - 6 * B2)],
                        conv_sh.at[pl.ds(r0 + 6 * B2, RW - 6 * B2)])
        plsc.subcore_barrier()

        coff = (cid * 2 + q) * N

        def _zds(j, _):
            dscat_v[1, pl.ds(j * 16, 16)] = jnp.zeros((16,), jnp.int32)
            return _
        lax.fori_loop(0, B2 // 16, _zds, None)
        pltpu.async_copy(msg_v.at[1], conv_sh.at[dscat_v.at[1]], ssem,
                         add=True)

        _fire_idx(0, 0)
        _wait_idx(0)
        _fire_gathers(0, 0, coff)
        _fire_idx(1, 1)

        def _pair(p, _):
            _slot(2 * p, 0, coff)
            _slot(2 * p + 1, 1, coff)
            return _
        lax.fori_loop(0, NB2 // 2, _pair, None)

        _wait_gathers(0)
        _wait_idx(1)
        pltpu.make_async_copy(
            msg_v.at[1], conv_sh.at[dscat_v.at[1]], ssem).wait()

        plsc.subcore_barrier()
        for k in range(6):
            pltpu.sync_copy(conv_sh.at[pl.ds(r0 + k * B2, B2)], msg_v.at[0])
            pltpu.sync_copy(msg_v.at[0],
                            conv_hbm.at[cid, q, pl.ds(r0 + k * B2, B2)])
        pltpu.sync_copy(conv_sh.at[pl.ds(r0 + 6 * B2, RW - 6 * B2)],
                        msg_v.at[0, pl.ds(0, RW - 6 * B2)])
        pltpu.sync_copy(msg_v.at[0, pl.ds(0, RW - 6 * B2)],
                        conv_hbm.at[cid, q, pl.ds(r0 + 6 * B2, RW - 6 * B2)])
        plsc.subcore_barrier()


def _pass2(src_e, dst_e, w_t, rcpd, xh_flat):
    f = pl.kernel(
        _pass2_body,
        out_type=jax.ShapeDtypeStruct((2, 2, NROW, Q), jnp.float32),
        mesh=_MESH,
        scratch_types=[
            pltpu.VMEM((2, B2), jnp.int32),
            pltpu.VMEM((2, B2), jnp.int32),
            pltpu.VMEM((2, B2), jnp.int32),
            pltpu.VMEM((2, B2, H * Q), jnp.float32),
            pltpu.VMEM((2, B2, 16), jnp.float32),
            pltpu.VMEM((2, B2, 16), jnp.float32),
            pltpu.VMEM((2, B2, Q), jnp.float32),
            pltpu.VMEM_SHARED((NROW, Q), jnp.float32),
            pltpu.SemaphoreType.DMA,
            pltpu.SemaphoreType.DMA,
            pltpu.SemaphoreType.DMA,
            pltpu.SemaphoreType.DMA,
            pltpu.SemaphoreType.DMA,
        ],
        compiler_params=pltpu.CompilerParams(use_tc_tiling_on_sc=False),
    )
    return f(src_e, dst_e, w_t, rcpd, xh_flat)


def _final_body(c0_ref, c1_ref, c2_ref, c3_ref, y_ref, bias_ref, out_ref):
    conv = jnp.concatenate(
        [c0_ref[...], c1_ref[...], c2_ref[...], c3_ref[...]],
        axis=-1) * (1.0 / H)
    out_ref[...] = jnp.maximum(conv + bias_ref[...] + y_ref[...], 0.0)


def _final(conv_p, y_out, bias):
    return pl.pallas_call(
        _final_body,
        grid=(_GRID,),
        in_specs=[
            pl.BlockSpec((_BLK_N, Q), lambda i: (i, 0)),
            pl.BlockSpec((_BLK_N, Q), lambda i: (i, 0)),
            pl.BlockSpec((_BLK_N, Q), lambda i: (i, 0)),
            pl.BlockSpec((_BLK_N, Q), lambda i: (i, 0)),
            pl.BlockSpec((_BLK_N, C), lambda i: (i, 4)),
            pl.BlockSpec((1, C), lambda i: (0, 0)),
        ],
        out_specs=pl.BlockSpec((_BLK_N, C), lambda i: (i, 0)),
        out_shape=jax.ShapeDtypeStruct((N, C), jnp.float32),
    )(conv_p[0, 0], conv_p[0, 1], conv_p[1, 0], conv_p[1, 1],
      y_out, bias.reshape(1, C))


def kernel(x, edge_index, W, att_src, att_dst, bias, W_lin):
    wcat = jnp.concatenate([W, W_lin.T], axis=1)
    hsel = (jnp.arange(16)[None, :] == (jnp.arange(H * C) // C)[:, None])
    attm_s = jnp.where(hsel, att_src.reshape(H * C)[:, None], 0.0)
    attm_d = jnp.where(hsel, att_dst.reshape(H * C)[:, None], 0.0)

    y_out, xh4, as_t, ad_t, shift = _dense_phase(x, wcat, attm_s, attm_d)

    as_p, ad_p = as_t, ad_t
    xh_flat = xh4.reshape(4 * N, H * Q)

    pad = EPAD - E - N
    loop = jnp.arange(N, dtype=jnp.int32)
    src_e = jnp.concatenate(
        [edge_index[0], loop, jnp.zeros((pad,), jnp.int32)])
    dst_e = jnp.concatenate(
        [edge_index[1], loop, jnp.full((pad,), N, jnp.int32)])

    w_t, dpart = _pass1(src_e, dst_e, as_p, ad_p, shift)
    rcpd = _dsum(dpart)
    conv_p = _pass2(src_e, dst_e, w_t, rcpd, xh_flat)
    return _final(conv_p, y_out, bias)
